# baseline (speedup 1.0000x reference)
import functools

import jax
import jax.numpy as jnp
from jax import lax
from jax.experimental import pallas as pl
from jax.experimental.pallas import tpu as pltpu
from jax.experimental.pallas import tpu_sc as plsc

VOCAB = 30000
EMB = 128
EPS = 1e-12
B = 128
S = 512

NC = 2
NS = 16
NW = NC * NS
ROWS = B * S
CHUNK = 128
GROUP = 2 * CHUNK


def _sc_gather(word_emb, ids2d):
    n_rows = ids2d.shape[0] * CHUNK
    rpw = n_rows // NW
    cpw = rpw // CHUNK
    ng = rpw // GROUP
    mesh = plsc.VectorSubcoreMesh(core_axis_name="c", subcore_axis_name="s")

    @functools.partial(
        pl.kernel,
        mesh=mesh,
        out_type=jax.ShapeDtypeStruct((n_rows, EMB), jnp.float32),
        scratch_types=[
            pltpu.VMEM((cpw, CHUNK), jnp.int32),
            pltpu.VMEM((GROUP, EMB), jnp.float32),
            pltpu.VMEM((GROUP, EMB), jnp.float32),
            pltpu.SemaphoreType.DMA,
            pltpu.SemaphoreType.DMA,
            pltpu.SemaphoreType.DMA,
            pltpu.SemaphoreType.DMA,
        ],
    )
    def k(table_hbm, idx_hbm, out_hbm, idx_v, buf0, buf1, g0, g1, w0, w1):
        wid = lax.axis_index("s") * NC + lax.axis_index("c")
        pltpu.sync_copy(idx_hbm.at[pl.ds(wid * cpw, cpw)], idx_v)
        bufs = (buf0, buf1)
        gsems = (g0, g1)
        wsems = (w0, w1)

        def fire(g):
            b = bufs[g % 2]
            sem = gsems[g % 2]
            return (
                pltpu.async_copy(table_hbm.at[idx_v.at[2 * g]],
                                 b.at[pl.ds(0, CHUNK)], sem),
                pltpu.async_copy(table_hbm.at[idx_v.at[2 * g + 1]],
                                 b.at[pl.ds(CHUNK, CHUNK)], sem),
            )

        writes = [None, None]
        pend = fire(0)
        for g in range(ng):
            if g + 1 < ng:
                if writes[(g + 1) % 2] is not None:
                    writes[(g + 1) % 2].wait()
                nxt = fire(g + 1)
            else:
                nxt = None
            pend[0].wait()
            pend[1].wait()
            writes[g % 2] = pltpu.async_copy(
                bufs[g % 2],
                out_hbm.at[pl.ds(wid * rpw + g * GROUP, GROUP)],
                wsems[g % 2])
            pend = nxt
        for w in writes:
            if w is not None:
                w.wait()

    return k(word_emb, ids2d)


BB = 8


def _tc_body(we_ref, tt_ref, pos_ref, dt_ref, g_ref, b_ref, prev_ref, out_ref):
    del prev_ref
    we = we_ref[...]
    ttf = tt_ref[...].reshape(BB, S, 1)
    dt = dt_ref[0][None, None, :]
    x = we + pos_ref[...][None] + ttf * dt
    mean = jnp.mean(x, axis=-1, keepdims=True)
    xc = x - mean
    var = jnp.mean(xc * xc, axis=-1, keepdims=True)
    y = xc * lax.rsqrt(var + EPS)
    out_ref[...] = y * g_ref[0][None, None, :] + b_ref[0][None, None, :]


def _tc_ln_part(we3, tt3, pos_emb, type_emb, gamma2, beta2, prev, block_off):
    nb = we3.shape[0] // BB
    in_specs = [
        pl.BlockSpec((BB, S, EMB), lambda i: (i, 0, 0)),
        pl.BlockSpec((BB, 1, S), lambda i: (i, 0, 0)),
        pl.BlockSpec((S, EMB), lambda i: (0, 0)),
        pl.BlockSpec((1, EMB), lambda i: (0, 0)),
        pl.BlockSpec((1, EMB), lambda i: (0, 0)),
        pl.BlockSpec((1, EMB), lambda i: (0, 0)),
    ]
    args = [we3, tt3, pos_emb, type_emb, gamma2, beta2]
    aliases = {}
    body = _tc_body
    if prev is not None:
        in_specs.append(pl.BlockSpec(memory_space=pl.ANY))
        args.append(prev)
        aliases = {6: 0}
    else:
        def body(we, tt, pos, ty, g, b, out):
            _tc_body(we, tt, pos, ty, g, b, None, out)
    return pl.pallas_call(
        body,
        grid=(nb,),
        in_specs=in_specs,
        out_specs=pl.BlockSpec((BB, S, EMB),
                               lambda i, o=block_off: (i + o, 0, 0)),
        out_shape=jax.ShapeDtypeStruct((B, S, EMB), jnp.float32),
        input_output_aliases=aliases,
        compiler_params=pltpu.CompilerParams(
            dimension_semantics=("parallel",)),
    )(*args)


def kernel(input_ids, position_ids, token_type_ids, inputs_embeds,
           word_emb, pos_emb, type_emb, ln_gamma, ln_beta):
    del position_ids, inputs_embeds
    ids2d = input_ids.reshape(ROWS // CHUNK, CHUNK)
    tt3 = token_type_ids.astype(jnp.float32).reshape(B, 1, S)
    gamma2 = ln_gamma.reshape(1, EMB)
    beta2 = ln_beta.reshape(1, EMB)
    posb = pos_emb + type_emb[0]
    dt2 = (type_emb[1] - type_emb[0]).reshape(1, EMB)

    nparts = 4
    hb = B // nparts
    hrows = ROWS // nparts
    hchunks = hrows // CHUNK
    out = None
    for p in range(nparts):
        rows = _sc_gather(word_emb, ids2d[p * hchunks:(p + 1) * hchunks])
        we3 = rows.reshape(hb, S, EMB)
        out = _tc_ln_part(we3, tt3[p * hb:(p + 1) * hb], posb, dt2,
                          gamma2, beta2, out, p * (hb // BB))
    return out


# speedup vs baseline: 1.0350x; 1.0350x over previous
import functools

import jax
import jax.numpy as jnp
from jax import lax
from jax.experimental import pallas as pl
from jax.experimental.pallas import tpu as pltpu
from jax.experimental.pallas import tpu_sc as plsc

VOCAB = 30000
EMB = 128
EPS = 1e-12
B = 128
S = 512

NC = 2
NS = 16
NW = NC * NS
ROWS = B * S
CHUNK = 128
GROUP = 2 * CHUNK


def _sc_gather(word_emb, ids2d):
    n_rows = ids2d.shape[0] * CHUNK
    rpw = n_rows // NW
    cpw = rpw // CHUNK
    ng = rpw // GROUP
    mesh = plsc.VectorSubcoreMesh(core_axis_name="c", subcore_axis_name="s")

    @functools.partial(
        pl.kernel,
        mesh=mesh,
        out_type=jax.ShapeDtypeStruct((n_rows, EMB), jnp.float32),
        scratch_types=[
            pltpu.VMEM((cpw, CHUNK), jnp.int32),
            pltpu.VMEM((GROUP, EMB), jnp.float32),
            pltpu.VMEM((GROUP, EMB), jnp.float32),
            pltpu.SemaphoreType.DMA,
            pltpu.SemaphoreType.DMA,
            pltpu.SemaphoreType.DMA,
            pltpu.SemaphoreType.DMA,
        ],
    )
    def k(table_hbm, idx_hbm, out_hbm, idx_v, buf0, buf1, g0, g1, w0, w1):
        wid = lax.axis_index("s") * NC + lax.axis_index("c")
        pltpu.sync_copy(idx_hbm.at[pl.ds(wid * cpw, cpw)], idx_v)
        bufs = (buf0, buf1)
        gsems = (g0, g1)
        wsems = (w0, w1)

        def fire(g):
            b = bufs[g % 2]
            sem = gsems[g % 2]
            return (
                pltpu.async_copy(table_hbm.at[idx_v.at[2 * g]],
                                 b.at[pl.ds(0, CHUNK)], sem),
                pltpu.async_copy(table_hbm.at[idx_v.at[2 * g + 1]],
                                 b.at[pl.ds(CHUNK, CHUNK)], sem),
            )

        writes = [None, None]
        pend = fire(0)
        for g in range(ng):
            if g + 1 < ng:
                if writes[(g + 1) % 2] is not None:
                    writes[(g + 1) % 2].wait()
                nxt = fire(g + 1)
            else:
                nxt = None
            pend[0].wait()
            pend[1].wait()
            writes[g % 2] = pltpu.async_copy(
                bufs[g % 2],
                out_hbm.at[pl.ds(wid * rpw + g * GROUP, GROUP)],
                wsems[g % 2])
            pend = nxt
        for w in writes:
            if w is not None:
                w.wait()

    return k(word_emb, ids2d)


BB = 8


def _tc_body(we_ref, tt_ref, pos_ref, dt_ref, g_ref, b_ref, prev_ref, out_ref):
    del prev_ref
    we = we_ref[...]
    ttf = tt_ref[...].reshape(BB, S, 1)
    dt = dt_ref[0][None, None, :]
    x = we + pos_ref[...][None] + ttf * dt
    mean = jnp.mean(x, axis=-1, keepdims=True)
    xc = x - mean
    var = jnp.mean(xc * xc, axis=-1, keepdims=True)
    y = xc * lax.rsqrt(var + EPS)
    out_ref[...] = y * g_ref[0][None, None, :] + b_ref[0][None, None, :]


def _tc_ln_part(we3, tt3, pos_emb, type_emb, gamma2, beta2, prev, block_off):
    nb = we3.shape[0] // BB
    in_specs = [
        pl.BlockSpec((BB, S, EMB), lambda i: (i, 0, 0)),
        pl.BlockSpec((BB, 1, S), lambda i: (i, 0, 0)),
        pl.BlockSpec((S, EMB), lambda i: (0, 0)),
        pl.BlockSpec((1, EMB), lambda i: (0, 0)),
        pl.BlockSpec((1, EMB), lambda i: (0, 0)),
        pl.BlockSpec((1, EMB), lambda i: (0, 0)),
    ]
    args = [we3, tt3, pos_emb, type_emb, gamma2, beta2]
    aliases = {}
    body = _tc_body
    if prev is not None:
        in_specs.append(pl.BlockSpec(memory_space=pl.ANY))
        args.append(prev)
        aliases = {6: 0}
    else:
        def body(we, tt, pos, ty, g, b, out):
            _tc_body(we, tt, pos, ty, g, b, None, out)
    return pl.pallas_call(
        body,
        grid=(nb,),
        in_specs=in_specs,
        out_specs=pl.BlockSpec((BB, S, EMB),
                               lambda i, o=block_off: (i + o, 0, 0)),
        out_shape=jax.ShapeDtypeStruct((B, S, EMB), jnp.float32),
        input_output_aliases=aliases,
        compiler_params=pltpu.CompilerParams(
            dimension_semantics=("parallel",)),
    )(*args)


def kernel(input_ids, position_ids, token_type_ids, inputs_embeds,
           word_emb, pos_emb, type_emb, ln_gamma, ln_beta):
    del position_ids, inputs_embeds
    ids2d = input_ids.reshape(ROWS // CHUNK, CHUNK)
    tt3 = token_type_ids.astype(jnp.float32).reshape(B, 1, S)
    gamma2 = ln_gamma.reshape(1, EMB)
    beta2 = ln_beta.reshape(1, EMB)
    posb = pos_emb + type_emb[0]
    dt2 = (type_emb[1] - type_emb[0]).reshape(1, EMB)

    nparts = 2
    hb = B // nparts
    hrows = ROWS // nparts
    hchunks = hrows // CHUNK
    out = None
    for p in range(nparts):
        rows = _sc_gather(word_emb, ids2d[p * hchunks:(p + 1) * hchunks])
        we3 = rows.reshape(hb, S, EMB)
        out = _tc_ln_part(we3, tt3[p * hb:(p + 1) * hb], posb, dt2,
                          gamma2, beta2, out, p * (hb // BB))
    return out


# speedup vs baseline: 1.0620x; 1.0260x over previous
import functools

import jax
import jax.numpy as jnp
from jax import lax
from jax.experimental import pallas as pl
from jax.experimental.pallas import tpu as pltpu
from jax.experimental.pallas import tpu_sc as plsc

VOCAB = 30000
EMB = 128
EPS = 1e-12
B = 128
S = 512

NC = 2
NS = 16
NW = NC * NS
ROWS = B * S
CHUNK = 128
GROUP = 2 * CHUNK


def _sc_gather(word_emb, ids2d):
    n_rows = ids2d.shape[0] * CHUNK
    rpw = n_rows // NW
    cpw = rpw // CHUNK
    ng = rpw // GROUP
    mesh = plsc.VectorSubcoreMesh(core_axis_name="c", subcore_axis_name="s")

    @functools.partial(
        pl.kernel,
        mesh=mesh,
        out_type=jax.ShapeDtypeStruct((n_rows, EMB), jnp.float32),
        scratch_types=[
            pltpu.VMEM((cpw, CHUNK), jnp.int32),
            pltpu.VMEM((GROUP, EMB), jnp.float32),
            pltpu.VMEM((GROUP, EMB), jnp.float32),
            pltpu.VMEM((GROUP, EMB), jnp.float32),
            pltpu.SemaphoreType.DMA,
            pltpu.SemaphoreType.DMA,
            pltpu.SemaphoreType.DMA,
            pltpu.SemaphoreType.DMA,
            pltpu.SemaphoreType.DMA,
            pltpu.SemaphoreType.DMA,
        ],
    )
    def k(table_hbm, idx_hbm, out_hbm, idx_v,
          buf0, buf1, buf2, g0, g1, g2, w0, w1, w2):
        wid = lax.axis_index("s") * NC + lax.axis_index("c")
        pltpu.sync_copy(idx_hbm.at[pl.ds(wid * cpw, cpw)], idx_v)
        bufs = (buf0, buf1, buf2)
        gsems = (g0, g1, g2)
        wsems = (w0, w1, w2)
        nbuf = len(bufs)

        pend = [None] * nbuf
        writes = [None] * nbuf

        def fire(g):
            b = g % nbuf
            if writes[b] is not None:
                writes[b].wait()
                writes[b] = None
            pend[b] = (
                pltpu.async_copy(table_hbm.at[idx_v.at[2 * g]],
                                 bufs[b].at[pl.ds(0, CHUNK)], gsems[b]),
                pltpu.async_copy(table_hbm.at[idx_v.at[2 * g + 1]],
                                 bufs[b].at[pl.ds(CHUNK, CHUNK)], gsems[b]),
            )

        for g in range(min(nbuf - 1, ng)):
            fire(g)
        for g in range(ng):
            if g + nbuf - 1 < ng:
                fire(g + nbuf - 1)
            b = g % nbuf
            pend[b][0].wait()
            pend[b][1].wait()
            writes[b] = pltpu.async_copy(
                bufs[b],
                out_hbm.at[pl.ds(wid * rpw + g * GROUP, GROUP)],
                wsems[b])
        for w in writes:
            if w is not None:
                w.wait()

    return k(word_emb, ids2d)


BB = 8


def _tc_body(we_ref, tt_ref, pos_ref, dt_ref, g_ref, b_ref, prev_ref, out_ref):
    del prev_ref
    we = we_ref[...]
    ttf = tt_ref[...].reshape(BB, S, 1)
    dt = dt_ref[0][None, None, :]
    x = we + pos_ref[...][None] + ttf * dt
    mean = jnp.mean(x, axis=-1, keepdims=True)
    xc = x - mean
    var = jnp.mean(xc * xc, axis=-1, keepdims=True)
    y = xc * lax.rsqrt(var + EPS)
    out_ref[...] = y * g_ref[0][None, None, :] + b_ref[0][None, None, :]


def _tc_ln_part(we3, tt3, pos_emb, type_emb, gamma2, beta2, prev, block_off):
    nb = we3.shape[0] // BB
    in_specs = [
        pl.BlockSpec((BB, S, EMB), lambda i: (i, 0, 0)),
        pl.BlockSpec((BB, 1, S), lambda i: (i, 0, 0)),
        pl.BlockSpec((S, EMB), lambda i: (0, 0)),
        pl.BlockSpec((1, EMB), lambda i: (0, 0)),
        pl.BlockSpec((1, EMB), lambda i: (0, 0)),
        pl.BlockSpec((1, EMB), lambda i: (0, 0)),
    ]
    args = [we3, tt3, pos_emb, type_emb, gamma2, beta2]
    aliases = {}
    body = _tc_body
    if prev is not None:
        in_specs.append(pl.BlockSpec(memory_space=pl.ANY))
        args.append(prev)
        aliases = {6: 0}
    else:
        def body(we, tt, pos, ty, g, b, out):
            _tc_body(we, tt, pos, ty, g, b, None, out)
    return pl.pallas_call(
        body,
        grid=(nb,),
        in_specs=in_specs,
        out_specs=pl.BlockSpec((BB, S, EMB),
                               lambda i, o=block_off: (i + o, 0, 0)),
        out_shape=jax.ShapeDtypeStruct((B, S, EMB), jnp.float32),
        input_output_aliases=aliases,
        compiler_params=pltpu.CompilerParams(
            dimension_semantics=("parallel",)),
    )(*args)


def kernel(input_ids, position_ids, token_type_ids, inputs_embeds,
           word_emb, pos_emb, type_emb, ln_gamma, ln_beta):
    del position_ids, inputs_embeds
    ids2d = input_ids.reshape(ROWS // CHUNK, CHUNK)
    tt3 = token_type_ids.astype(jnp.float32).reshape(B, 1, S)
    gamma2 = ln_gamma.reshape(1, EMB)
    beta2 = ln_beta.reshape(1, EMB)
    posb = pos_emb + type_emb[0]
    dt2 = (type_emb[1] - type_emb[0]).reshape(1, EMB)

    nparts = 2
    hb = B // nparts
    hrows = ROWS // nparts
    hchunks = hrows // CHUNK
    out = None
    for p in range(nparts):
        rows = _sc_gather(word_emb, ids2d[p * hchunks:(p + 1) * hchunks])
        we3 = rows.reshape(hb, S, EMB)
        out = _tc_ln_part(we3, tt3[p * hb:(p + 1) * hb], posb, dt2,
                          gamma2, beta2, out, p * (hb // BB))
    return out


# speedup vs baseline: 1.0667x; 1.0044x over previous
import functools

import jax
import jax.numpy as jnp
from jax import lax
from jax.experimental import pallas as pl
from jax.experimental.pallas import tpu as pltpu
from jax.experimental.pallas import tpu_sc as plsc

VOCAB = 30000
EMB = 128
EPS = 1e-12
B = 128
S = 512

NC = 2
NS = 16
NW = NC * NS
ROWS = B * S
CHUNK = 128
GROUP = 2 * CHUNK


def _sc_gather(word_emb, ids2d):
    n_rows = ids2d.shape[0] * CHUNK
    rpw = n_rows // NW
    cpw = rpw // CHUNK
    ng = rpw // GROUP
    mesh = plsc.VectorSubcoreMesh(core_axis_name="c", subcore_axis_name="s")

    @functools.partial(
        pl.kernel,
        mesh=mesh,
        out_type=jax.ShapeDtypeStruct((n_rows, EMB), jnp.float32),
        scratch_types=[
            pltpu.VMEM((cpw, CHUNK), jnp.int32),
            pltpu.VMEM((GROUP, EMB), jnp.float32),
            pltpu.VMEM((GROUP, EMB), jnp.float32),
            pltpu.VMEM((GROUP, EMB), jnp.float32),
            pltpu.SemaphoreType.DMA,
            pltpu.SemaphoreType.DMA,
            pltpu.SemaphoreType.DMA,
            pltpu.SemaphoreType.DMA,
            pltpu.SemaphoreType.DMA,
            pltpu.SemaphoreType.DMA,
        ],
    )
    def k(table_hbm, idx_hbm, out_hbm, idx_v,
          buf0, buf1, buf2, g0, g1, g2, w0, w1, w2):
        wid = lax.axis_index("s") * NC + lax.axis_index("c")
        pltpu.sync_copy(idx_hbm.at[pl.ds(wid * cpw, cpw)], idx_v)
        bufs = (buf0, buf1, buf2)
        gsems = (g0, g1, g2)
        wsems = (w0, w1, w2)
        nbuf = len(bufs)

        pend = [None] * nbuf
        writes = [None] * nbuf

        def fire(g):
            b = g % nbuf
            if writes[b] is not None:
                writes[b].wait()
                writes[b] = None
            pend[b] = (
                pltpu.async_copy(table_hbm.at[idx_v.at[2 * g]],
                                 bufs[b].at[pl.ds(0, CHUNK)], gsems[b]),
                pltpu.async_copy(table_hbm.at[idx_v.at[2 * g + 1]],
                                 bufs[b].at[pl.ds(CHUNK, CHUNK)], gsems[b]),
            )

        for g in range(min(nbuf - 1, ng)):
            fire(g)
        for g in range(ng):
            if g + nbuf - 1 < ng:
                fire(g + nbuf - 1)
            b = g % nbuf
            pend[b][0].wait()
            pend[b][1].wait()
            writes[b] = pltpu.async_copy(
                bufs[b],
                out_hbm.at[pl.ds(wid * rpw + g * GROUP, GROUP)],
                wsems[b])
        for w in writes:
            if w is not None:
                w.wait()

    return k(word_emb, ids2d)


BB = 16


def _tc_body(we_ref, tt_ref, pos_ref, dt_ref, g_ref, b_ref, prev_ref, out_ref):
    del prev_ref
    we = we_ref[...]
    bb = we_ref.shape[0]
    ttf = tt_ref[...].reshape(bb, S, 1)
    dt = dt_ref[0][None, None, :]
    x = we + pos_ref[...][None] + ttf * dt
    mean = jnp.mean(x, axis=-1, keepdims=True)
    xc = x - mean
    var = jnp.mean(xc * xc, axis=-1, keepdims=True)
    y = xc * lax.rsqrt(var + EPS)
    out_ref[...] = y * g_ref[0][None, None, :] + b_ref[0][None, None, :]


def _tc_ln_part(we3, tt3, pos_emb, type_emb, gamma2, beta2, prev, block_off):
    nb = we3.shape[0] // BB
    in_specs = [
        pl.BlockSpec((BB, S, EMB), lambda i: (i, 0, 0)),
        pl.BlockSpec((BB, 1, S), lambda i: (i, 0, 0)),
        pl.BlockSpec((S, EMB), lambda i: (0, 0)),
        pl.BlockSpec((1, EMB), lambda i: (0, 0)),
        pl.BlockSpec((1, EMB), lambda i: (0, 0)),
        pl.BlockSpec((1, EMB), lambda i: (0, 0)),
    ]
    args = [we3, tt3, pos_emb, type_emb, gamma2, beta2]
    aliases = {}
    body = _tc_body
    if prev is not None:
        in_specs.append(pl.BlockSpec(memory_space=pl.ANY))
        args.append(prev)
        aliases = {6: 0}
    else:
        def body(we, tt, pos, ty, g, b, out):
            _tc_body(we, tt, pos, ty, g, b, None, out)
    return pl.pallas_call(
        body,
        grid=(nb,),
        in_specs=in_specs,
        out_specs=pl.BlockSpec((BB, S, EMB),
                               lambda i, o=block_off: (i + o, 0, 0)),
        out_shape=jax.ShapeDtypeStruct((B, S, EMB), jnp.float32),
        input_output_aliases=aliases,
        compiler_params=pltpu.CompilerParams(
            dimension_semantics=("parallel",)),
    )(*args)


def kernel(input_ids, position_ids, token_type_ids, inputs_embeds,
           word_emb, pos_emb, type_emb, ln_gamma, ln_beta):
    del position_ids, inputs_embeds
    ids2d = input_ids.reshape(ROWS // CHUNK, CHUNK)
    tt3 = token_type_ids.astype(jnp.float32).reshape(B, 1, S)
    gamma2 = ln_gamma.reshape(1, EMB)
    beta2 = ln_beta.reshape(1, EMB)
    posb = pos_emb + type_emb[0]
    dt2 = (type_emb[1] - type_emb[0]).reshape(1, EMB)

    nparts = 2
    hb = B // nparts
    hrows = ROWS // nparts
    hchunks = hrows // CHUNK
    out = None
    for p in range(nparts):
        rows = _sc_gather(word_emb, ids2d[p * hchunks:(p + 1) * hchunks])
        we3 = rows.reshape(hb, S, EMB)
        out = _tc_ln_part(we3, tt3[p * hb:(p + 1) * hb], posb, dt2,
                          gamma2, beta2, out, p * (hb // BB))
    return out


# speedup vs baseline: 1.0673x; 1.0006x over previous
import functools

import jax
import jax.numpy as jnp
from jax import lax
from jax.experimental import pallas as pl
from jax.experimental.pallas import tpu as pltpu
from jax.experimental.pallas import tpu_sc as plsc

VOCAB = 30000
EMB = 128
EPS = 1e-12
B = 128
S = 512

NC = 2
NS = 16
NW = NC * NS
ROWS = B * S
CHUNK = 128
GROUP = CHUNK


def _sc_gather(word_emb, ids2d):
    n_rows = ids2d.shape[0] * CHUNK
    rpw = n_rows // NW
    cpw = rpw // CHUNK
    ng = rpw // GROUP
    mesh = plsc.VectorSubcoreMesh(core_axis_name="c", subcore_axis_name="s")

    @functools.partial(
        pl.kernel,
        mesh=mesh,
        out_type=jax.ShapeDtypeStruct((n_rows, EMB), jnp.float32),
        scratch_types=[
            pltpu.VMEM((cpw, CHUNK), jnp.int32),
            pltpu.VMEM((GROUP, EMB), jnp.float32),
            pltpu.VMEM((GROUP, EMB), jnp.float32),
            pltpu.VMEM((GROUP, EMB), jnp.float32),
            pltpu.VMEM((GROUP, EMB), jnp.float32),
            pltpu.SemaphoreType.DMA,
            pltpu.SemaphoreType.DMA,
            pltpu.SemaphoreType.DMA,
            pltpu.SemaphoreType.DMA,
            pltpu.SemaphoreType.DMA,
            pltpu.SemaphoreType.DMA,
            pltpu.SemaphoreType.DMA,
            pltpu.SemaphoreType.DMA,
        ],
    )
    def k(table_hbm, idx_hbm, out_hbm, idx_v,
          buf0, buf1, buf2, buf3, g0, g1, g2, g3, w0, w1, w2, w3):
        wid = lax.axis_index("s") * NC + lax.axis_index("c")
        pltpu.sync_copy(idx_hbm.at[pl.ds(wid * cpw, cpw)], idx_v)
        bufs = (buf0, buf1, buf2, buf3)
        gsems = (g0, g1, g2, g3)
        wsems = (w0, w1, w2, w3)
        nbuf = len(bufs)

        pend = [None] * nbuf
        writes = [None] * nbuf

        def fire(g):
            b = g % nbuf
            if writes[b] is not None:
                writes[b].wait()
                writes[b] = None
            pend[b] = pltpu.async_copy(
                table_hbm.at[idx_v.at[g]], bufs[b], gsems[b])

        for g in range(min(nbuf - 1, ng)):
            fire(g)
        for g in range(ng):
            if g + nbuf - 1 < ng:
                fire(g + nbuf - 1)
            b = g % nbuf
            pend[b].wait()
            writes[b] = pltpu.async_copy(
                bufs[b],
                out_hbm.at[pl.ds(wid * rpw + g * GROUP, GROUP)],
                wsems[b])
        for w in writes:
            if w is not None:
                w.wait()

    return k(word_emb, ids2d)


BB = 16


def _tc_body(we_ref, tt_ref, pos_ref, dt_ref, g_ref, b_ref, prev_ref, out_ref):
    del prev_ref
    we = we_ref[...]
    bb = we_ref.shape[0]
    ttf = tt_ref[...].reshape(bb, S, 1)
    dt = dt_ref[0][None, None, :]
    x = we + pos_ref[...][None] + ttf * dt
    mean = jnp.mean(x, axis=-1, keepdims=True)
    xc = x - mean
    var = jnp.mean(xc * xc, axis=-1, keepdims=True)
    y = xc * lax.rsqrt(var + EPS)
    out_ref[...] = y * g_ref[0][None, None, :] + b_ref[0][None, None, :]


def _tc_ln_part(we3, tt3, pos_emb, type_emb, gamma2, beta2, prev, block_off):
    nb = we3.shape[0] // BB
    in_specs = [
        pl.BlockSpec((BB, S, EMB), lambda i: (i, 0, 0)),
        pl.BlockSpec((BB, 1, S), lambda i: (i, 0, 0)),
        pl.BlockSpec((S, EMB), lambda i: (0, 0)),
        pl.BlockSpec((1, EMB), lambda i: (0, 0)),
        pl.BlockSpec((1, EMB), lambda i: (0, 0)),
        pl.BlockSpec((1, EMB), lambda i: (0, 0)),
    ]
    args = [we3, tt3, pos_emb, type_emb, gamma2, beta2]
    aliases = {}
    body = _tc_body
    if prev is not None:
        in_specs.append(pl.BlockSpec(memory_space=pl.ANY))
        args.append(prev)
        aliases = {6: 0}
    else:
        def body(we, tt, pos, ty, g, b, out):
            _tc_body(we, tt, pos, ty, g, b, None, out)
    return pl.pallas_call(
        body,
        grid=(nb,),
        in_specs=in_specs,
        out_specs=pl.BlockSpec((BB, S, EMB),
                               lambda i, o=block_off: (i + o, 0, 0)),
        out_shape=jax.ShapeDtypeStruct((B, S, EMB), jnp.float32),
        input_output_aliases=aliases,
        compiler_params=pltpu.CompilerParams(
            dimension_semantics=("parallel",)),
    )(*args)


def kernel(input_ids, position_ids, token_type_ids, inputs_embeds,
           word_emb, pos_emb, type_emb, ln_gamma, ln_beta):
    del position_ids, inputs_embeds
    ids2d = input_ids.reshape(ROWS // CHUNK, CHUNK)
    tt3 = token_type_ids.astype(jnp.float32).reshape(B, 1, S)
    gamma2 = ln_gamma.reshape(1, EMB)
    beta2 = ln_beta.reshape(1, EMB)
    posb = pos_emb + type_emb[0]
    dt2 = (type_emb[1] - type_emb[0]).reshape(1, EMB)

    nparts = 2
    hb = B // nparts
    hrows = ROWS // nparts
    hchunks = hrows // CHUNK
    out = None
    for p in range(nparts):
        rows = _sc_gather(word_emb, ids2d[p * hchunks:(p + 1) * hchunks])
        we3 = rows.reshape(hb, S, EMB)
        out = _tc_ln_part(we3, tt3[p * hb:(p + 1) * hb], posb, dt2,
                          gamma2, beta2, out, p * (hb // BB))
    return out
